# baseline (speedup 1.0000x reference)
import functools

import jax
import jax.numpy as jnp
from jax import lax
from jax.experimental import pallas as pl
from jax.experimental.pallas import tpu as pltpu
from jax.experimental.pallas import tpu_sc as plsc

NF = 384
G = 6
NG = 192 // G


def kernel(x, indices):
    B, C, H, W = x.shape
    zplanes = jnp.zeros((32, G, H, W), x.dtype)

    mesh = plsc.VectorSubcoreMesh(core_axis_name="c", subcore_axis_name="s")

    @functools.partial(
        pl.kernel,
        out_type=jax.ShapeDtypeStruct((B, NF, H, W), x.dtype),
        mesh=mesh,
        scratch_types=[
            pltpu.VMEM((208,), jnp.int32),
            pltpu.VMEM((2, G, H, W), x.dtype),
            pltpu.VMEM((G, H, W), x.dtype),
            pltpu.SemaphoreType.DMA,
            pltpu.SemaphoreType.DMA,
            pltpu.SemaphoreType.DMA,
            pltpu.SemaphoreType.DMA,
        ],
        compiler_params=pltpu.CompilerParams(
            use_tc_tiling_on_sc=False, needs_layout_passes=False
        ),
    )
    def sc_gather(x_hbm, idx_hbm, z_hbm, out_hbm, idx_v, buf_v, zbank_v,
                  gsem0, gsem1, ssem0, ssem1):
        gsems = (gsem0, gsem1)
        ssems = (ssem0, ssem1)
        wid = lax.axis_index("c") * 16 + lax.axis_index("s")
        b = wid // 2
        i0 = (wid % 2) * 192

        pltpu.sync_copy(idx_hbm.at[pl.ds(i0, 192)], idx_v.at[pl.ds(0, 192)])
        pltpu.sync_copy(z_hbm.at[wid], zbank_v)

        lanes = lax.iota(jnp.int32, 16) < G

        def drain6(sem):
            pltpu.make_async_copy(x_hbm.at[b, pl.ds(0, G)], buf_v.at[0],
                                  sem).wait()

        def fire_gathers(g, bank):
            iv = idx_v[pl.ds(g * G, 16)]
            v0 = iv[0]
            valid = (iv < C) & lanes
            nv = plsc.all_reduce_population_count(valid)[0]
            consec = (iv == v0 + lax.iota(jnp.int32, 16)) & lanes
            fast = (nv == G) & (plsc.all_reduce_population_count(consec)[0] == G)
            allpad = nv == 0

            @pl.when(fast)
            def _big(v0=v0, bank=bank):
                pltpu.async_copy(x_hbm.at[b, pl.ds(v0, G)], buf_v.at[bank],
                                 gsems[bank])

            @pl.when(jnp.logical_not(fast) & jnp.logical_not(allpad))
            def _slow(iv=iv, bank=bank):
                for j in range(G):
                    v = iv[j]

                    @pl.when(v < C)
                    def _plane(j=j, v=v, bank=bank):
                        pltpu.async_copy(x_hbm.at[b, v], buf_v.at[bank, j],
                                         gsems[bank])

                    @pl.when(v >= C)
                    def _fill(j=j, bank=bank):
                        pltpu.async_copy(z_hbm.at[wid, j], buf_v.at[bank, j],
                                         gsems[bank])

            return allpad

        def fire_store(g, bank, allpad):
            ch0 = i0 + g * G

            @pl.when(allpad)
            def _z(ch0=ch0, bank=bank):
                pltpu.async_copy(zbank_v, out_hbm.at[b, pl.ds(ch0, G)],
                                 ssems[bank])

            @pl.when(jnp.logical_not(allpad))
            def _d(ch0=ch0, bank=bank):
                drain6(gsems[bank])
                pltpu.async_copy(buf_v.at[bank], out_hbm.at[b, pl.ds(ch0, G)],
                                 ssems[bank])

        pads = [None] * NG
        pads[0] = fire_gathers(0, 0)
        for g in range(1, NG):
            bank = g & 1
            if g >= 2:
                drain6(ssems[bank])
            pads[g] = fire_gathers(g, bank)
            fire_store(g - 1, 1 - bank, pads[g - 1])
        fire_store(NG - 1, (NG - 1) & 1, pads[NG - 1])
        drain6(ssems[0])
        drain6(ssems[1])

    return sc_gather(x, indices, zplanes)


# speedup vs baseline: 1.4017x; 1.4017x over previous
import functools

import jax
import jax.numpy as jnp
from jax import lax
from jax.experimental import pallas as pl
from jax.experimental.pallas import tpu as pltpu
from jax.experimental.pallas import tpu_sc as plsc

NF = 384
G = 6
NG = 192 // G


def kernel(x, indices):
    B, C, H, W = x.shape
    zplanes = jnp.zeros((32, G, H, W), x.dtype)

    mesh = plsc.VectorSubcoreMesh(core_axis_name="c", subcore_axis_name="s")

    @functools.partial(
        pl.kernel,
        out_type=jax.ShapeDtypeStruct((B, NF, H, W), x.dtype),
        mesh=mesh,
        scratch_types=[
            pltpu.VMEM((208,), jnp.int32),
            pltpu.VMEM((2, G, H, W), x.dtype),
            pltpu.VMEM((G, H, W), x.dtype),
            pltpu.SemaphoreType.DMA,
            pltpu.SemaphoreType.DMA,
            pltpu.SemaphoreType.DMA,
            pltpu.SemaphoreType.DMA,
        ],
        compiler_params=pltpu.CompilerParams(
            use_tc_tiling_on_sc=True, needs_layout_passes=False
        ),
    )
    def sc_gather(x_hbm, idx_hbm, z_hbm, out_hbm, idx_v, buf_v, zbank_v,
                  gsem0, gsem1, ssem0, ssem1):
        gsems = (gsem0, gsem1)
        ssems = (ssem0, ssem1)
        wid = lax.axis_index("c") * 16 + lax.axis_index("s")
        b = wid // 2
        i0 = (wid % 2) * 192

        pltpu.sync_copy(idx_hbm.at[pl.ds(i0, 192)], idx_v.at[pl.ds(0, 192)])
        pltpu.sync_copy(z_hbm.at[wid], zbank_v)

        lanes = lax.iota(jnp.int32, 16) < G

        def drain6(sem):
            pltpu.make_async_copy(x_hbm.at[b, pl.ds(0, G)], buf_v.at[0],
                                  sem).wait()

        def fire_gathers(g, bank):
            iv = idx_v[pl.ds(g * G, 16)]
            v0 = iv[0]
            valid = (iv < C) & lanes
            nv = plsc.all_reduce_population_count(valid)[0]
            consec = (iv == v0 + lax.iota(jnp.int32, 16)) & lanes
            fast = (nv == G) & (plsc.all_reduce_population_count(consec)[0] == G)
            allpad = nv == 0

            @pl.when(fast)
            def _big():
                pltpu.async_copy(x_hbm.at[b, pl.ds(v0, G)], buf_v.at[bank],
                                 gsems[bank])

            @pl.when(jnp.logical_not(fast) & jnp.logical_not(allpad))
            def _slow():
                for j in range(G):
                    v = iv[j]

                    @pl.when(v < C)
                    def _plane(j=j, v=v):
                        pltpu.async_copy(x_hbm.at[b, v], buf_v.at[bank, j],
                                         gsems[bank])

                    @pl.when(v >= C)
                    def _fill(j=j):
                        pltpu.async_copy(z_hbm.at[wid, j], buf_v.at[bank, j],
                                         gsems[bank])

            return allpad

        def fire_store(g, bank, allpad):
            ch0 = i0 + g * G

            @pl.when(allpad)
            def _z():
                pltpu.async_copy(zbank_v, out_hbm.at[b, pl.ds(ch0, G)],
                                 ssems[bank])

            @pl.when(jnp.logical_not(allpad))
            def _d():
                drain6(gsems[bank])
                pltpu.async_copy(buf_v.at[bank], out_hbm.at[b, pl.ds(ch0, G)],
                                 ssems[bank])

        pad0 = fire_gathers(0, 0)
        pad1 = fire_gathers(1, 1)
        fire_store(0, 0, pad0)

        def pair(p, pad_prev):
            gA = 2 * p
            drain6(ssems[0])
            padA = fire_gathers(gA, 0)
            fire_store(gA - 1, 1, pad_prev)
            drain6(ssems[1])
            padB = fire_gathers(gA + 1, 1)
            fire_store(gA, 0, padA)
            return padB

        pad_last = lax.fori_loop(1, NG // 2, pair, pad1)
        fire_store(NG - 1, 1, pad_last)
        drain6(ssems[0])
        drain6(ssems[1])

    return sc_gather(x, indices, zplanes)


# speedup vs baseline: 1.8346x; 1.3088x over previous
import functools

import jax
import jax.numpy as jnp
from jax import lax
from jax.experimental import pallas as pl
from jax.experimental.pallas import tpu as pltpu
from jax.experimental.pallas import tpu_sc as plsc

NF = 384
G = 6
NG = 192 // G


def kernel(x, indices):
    B, C, H, W = x.shape
    x3 = x.reshape(B * C, H, W)
    zplanes = jnp.zeros((32, G, H, W), x.dtype)

    mesh = plsc.VectorSubcoreMesh(core_axis_name="c", subcore_axis_name="s")

    @functools.partial(
        pl.kernel,
        out_type=jax.ShapeDtypeStruct((B * NF, H, W), x.dtype),
        mesh=mesh,
        scratch_types=[
            pltpu.VMEM((208,), jnp.int32),
            pltpu.VMEM((2, G, H, W), x.dtype),
            pltpu.VMEM((G, H, W), x.dtype),
            pltpu.SemaphoreType.DMA,
            pltpu.SemaphoreType.DMA,
            pltpu.SemaphoreType.DMA,
            pltpu.SemaphoreType.DMA,
        ],
        compiler_params=pltpu.CompilerParams(
            use_tc_tiling_on_sc=True, needs_layout_passes=False
        ),
    )
    def sc_gather(x_hbm, idx_hbm, z_hbm, out_hbm, idx_v, buf_v, zbank_v,
                  gsem0, gsem1, ssem0, ssem1):
        gsems = (gsem0, gsem1)
        ssems = (ssem0, ssem1)
        wid = lax.axis_index("c") * 16 + lax.axis_index("s")
        b = wid // 2
        i0 = (wid % 2) * 192
        bC = b * C
        ob = b * NF

        pltpu.sync_copy(idx_hbm.at[pl.ds(i0, 192)], idx_v.at[pl.ds(0, 192)])
        pltpu.sync_copy(z_hbm.at[wid], zbank_v)

        lanes = lax.iota(jnp.int32, 16) < G

        def drain6(sem):
            pltpu.make_async_copy(x_hbm.at[pl.ds(0, G)], buf_v.at[0],
                                  sem).wait()

        def fire_gathers(g, bank):
            iv = idx_v[pl.ds(g * G, 16)]
            v0 = iv[0]
            valid = (iv < C) & lanes
            nv = plsc.all_reduce_population_count(valid)[0]
            consec = (iv == v0 + lax.iota(jnp.int32, 16)) & lanes
            fast = (nv == G) & (plsc.all_reduce_population_count(consec)[0] == G)
            allpad = nv == 0

            @pl.when(fast)
            def _big():
                pltpu.async_copy(x_hbm.at[pl.ds(bC + v0, G)], buf_v.at[bank],
                                 gsems[bank])

            @pl.when(jnp.logical_not(fast) & jnp.logical_not(allpad))
            def _slow():
                for j in range(G):
                    v = iv[j]

                    @pl.when(v < C)
                    def _plane(j=j, v=v):
                        pltpu.async_copy(x_hbm.at[bC + v], buf_v.at[bank, j],
                                         gsems[bank])

                    @pl.when(v >= C)
                    def _fill(j=j):
                        pltpu.async_copy(z_hbm.at[wid, j], buf_v.at[bank, j],
                                         gsems[bank])

            return allpad

        def fire_store(g, bank, allpad):
            ch0 = i0 + g * G

            @pl.when(allpad)
            def _z():
                pltpu.async_copy(zbank_v, out_hbm.at[pl.ds(ob + ch0, G)],
                                 ssems[bank])

            @pl.when(jnp.logical_not(allpad))
            def _d():
                drain6(gsems[bank])
                pltpu.async_copy(buf_v.at[bank], out_hbm.at[pl.ds(ob + ch0, G)],
                                 ssems[bank])

        pad0 = fire_gathers(0, 0)
        pad1 = fire_gathers(1, 1)
        fire_store(0, 0, pad0)

        def pair(p, pad_prev):
            gA = 2 * p
            drain6(ssems[0])
            padA = fire_gathers(gA, 0)
            fire_store(gA - 1, 1, pad_prev)
            drain6(ssems[1])
            padB = fire_gathers(gA + 1, 1)
            fire_store(gA, 0, padA)
            return padB

        pad_last = lax.fori_loop(1, NG // 2, pair, pad1)
        fire_store(NG - 1, 1, pad_last)
        drain6(ssems[0])
        drain6(ssems[1])

    out = sc_gather(x3, indices, zplanes)
    return out.reshape(B, NF, H, W)


# speedup vs baseline: 1.8371x; 1.0014x over previous
import functools

import jax
import jax.numpy as jnp
from jax import lax
from jax.experimental import pallas as pl
from jax.experimental.pallas import tpu as pltpu
from jax.experimental.pallas import tpu_sc as plsc

NF = 384
G = 6
NG = 192 // G


def kernel(x, indices):
    B, C, H, W = x.shape
    x3 = x.reshape(B * C, H, W)
    zplanes = jnp.zeros((32, G, H, W), x.dtype)

    mesh = plsc.VectorSubcoreMesh(core_axis_name="c", subcore_axis_name="s")

    @functools.partial(
        pl.kernel,
        out_type=jax.ShapeDtypeStruct((B * NF, H, W), x.dtype),
        mesh=mesh,
        scratch_types=[
            pltpu.VMEM((208,), jnp.int32),
            pltpu.VMEM((2, G, H, W), x.dtype),
            pltpu.VMEM((G, H, W), x.dtype),
            pltpu.SemaphoreType.DMA,
            pltpu.SemaphoreType.DMA,
            pltpu.SemaphoreType.DMA,
            pltpu.SemaphoreType.DMA,
        ],
        compiler_params=pltpu.CompilerParams(
            use_tc_tiling_on_sc=True, needs_layout_passes=False
        ),
    )
    def sc_gather(x_hbm, idx_hbm, z_hbm, out_hbm, idx_v, buf_v, zbank_v,
                  gsem0, gsem1, ssem0, ssem1):
        gsems = (gsem0, gsem1)
        ssems = (ssem0, ssem1)
        wid = lax.axis_index("c") * 16 + lax.axis_index("s")
        b = wid // 2
        i0 = (wid % 2) * 192
        bC = b * C
        ob = b * NF

        pltpu.sync_copy(idx_hbm.at[pl.ds(i0, 192)], idx_v.at[pl.ds(0, 192)])
        pltpu.sync_copy(z_hbm.at[wid], zbank_v)

        lanes = lax.iota(jnp.int32, 16) < G

        def drain6(sem):
            pltpu.make_async_copy(x_hbm.at[pl.ds(0, G)], buf_v.at[0],
                                  sem).wait()

        def fire_gathers(g, bank):
            iv = idx_v[pl.ds(g * G, 16)]
            v0 = iv[0]
            valid = (iv < C) & lanes
            nv = plsc.all_reduce_population_count(valid)[0]
            consec = (iv == v0 + lax.iota(jnp.int32, 16)) & lanes
            fast = (nv == G) & (plsc.all_reduce_population_count(consec)[0] == G) & jnp.bool_(False)
            allpad = nv == 0

            @pl.when(fast)
            def _big():
                pltpu.async_copy(x_hbm.at[pl.ds(bC + v0, G)], buf_v.at[bank],
                                 gsems[bank])

            @pl.when(jnp.logical_not(fast) & jnp.logical_not(allpad))
            def _slow():
                for j in range(G):
                    v = iv[j]

                    @pl.when(v < C)
                    def _plane(j=j, v=v):
                        pltpu.async_copy(x_hbm.at[bC + v], buf_v.at[bank, j],
                                         gsems[bank])

                    @pl.when(v >= C)
                    def _fill(j=j):
                        pltpu.async_copy(z_hbm.at[wid, j], buf_v.at[bank, j],
                                         gsems[bank])

            return allpad

        def fire_store(g, bank, allpad):
            ch0 = i0 + g * G

            @pl.when(allpad)
            def _z():
                pltpu.async_copy(zbank_v, out_hbm.at[pl.ds(ob + ch0, G)],
                                 ssems[bank])

            @pl.when(jnp.logical_not(allpad))
            def _d():
                drain6(gsems[bank])
                pltpu.async_copy(buf_v.at[bank], out_hbm.at[pl.ds(ob + ch0, G)],
                                 ssems[bank])

        pad0 = fire_gathers(0, 0)
        pad1 = fire_gathers(1, 1)
        fire_store(0, 0, pad0)

        def pair(p, pad_prev):
            gA = 2 * p
            drain6(ssems[0])
            padA = fire_gathers(gA, 0)
            fire_store(gA - 1, 1, pad_prev)
            drain6(ssems[1])
            padB = fire_gathers(gA + 1, 1)
            fire_store(gA, 0, padA)
            return padB

        pad_last = lax.fori_loop(1, NG // 2, pair, pad1)
        fire_store(NG - 1, 1, pad_last)
        drain6(ssems[0])
        drain6(ssems[1])

    out = sc_gather(x3, indices, zplanes)
    return out.reshape(B, NF, H, W)
